# Initial kernel scaffold; baseline (speedup 1.0000x reference)
import functools

import jax
import jax.numpy as jnp
from jax import lax
from jax.experimental import pallas as pl
from jax.experimental.pallas import tpu as pltpu
from jax.experimental.pallas import tpu_sc as plsc

N = 10000
E = 320000
CH = 128
NC, NS, L = 2, 16, 16
NW = NC * NS
EPW = E // NW
CHUNK = 80
NCHUNK = EPW // CHUNK
ROWS_PT = N // NS
DEN_PAD = 10240
DEN_PT = DEN_PAD // NS
ZROWS = 125

_mesh = plsc.VectorSubcoreMesh(
    core_axis_name="c", subcore_axis_name="s", num_cores=NC, num_subcores=NS
)


def _proj_body(x_ref, w_ref, asv_ref, adv_ref,
               xp_ref, as_ref, ad_ref, c_ref, ms_ref, md_ref):
    i = pl.program_id(0)
    xp = jnp.dot(x_ref[...], w_ref[...], preferred_element_type=jnp.float32)
    xp_ref[...] = xp
    s = jnp.sum(xp * asv_ref[...], axis=1, keepdims=True)
    d = jnp.sum(xp * adv_ref[...], axis=1, keepdims=True)
    as_ref[...] = s
    ad_ref[...] = d

    @pl.when(i == 0)
    def _():
        ms_ref[0] = -jnp.inf
        md_ref[0] = -jnp.inf

    ms_ref[0] = jnp.maximum(ms_ref[0], jnp.max(s))
    md_ref[0] = jnp.maximum(md_ref[0], jnp.max(d))

    @pl.when(i == pl.num_programs(0) - 1)
    def _():
        m = ms_ref[0] + md_ref[0]
        c_ref[0, 0] = jnp.where(m >= 0.0, m, 0.2 * m)


_PB = 400


def _project(x, W, a_src_v, a_dst_v):
    return pl.pallas_call(
        _proj_body,
        grid=(N // _PB,),
        in_specs=[
            pl.BlockSpec((_PB, CH), lambda i: (i, 0)),
            pl.BlockSpec((CH, CH), lambda i: (0, 0)),
            pl.BlockSpec((1, CH), lambda i: (0, 0)),
            pl.BlockSpec((1, CH), lambda i: (0, 0)),
        ],
        out_specs=[
            pl.BlockSpec((_PB, CH), lambda i: (i, 0)),
            pl.BlockSpec((_PB, 1), lambda i: (i, 0)),
            pl.BlockSpec((_PB, 1), lambda i: (i, 0)),
            pl.BlockSpec((1, 1), lambda i: (0, 0)),
        ],
        out_shape=[
            jax.ShapeDtypeStruct((N, CH), jnp.float32),
            jax.ShapeDtypeStruct((N, 1), jnp.float32),
            jax.ShapeDtypeStruct((N, 1), jnp.float32),
            jax.ShapeDtypeStruct((1, 1), jnp.float32),
        ],
        scratch_shapes=[
            pltpu.SMEM((1,), jnp.float32),
            pltpu.SMEM((1,), jnp.float32),
        ],
    )(x, W, a_src_v, a_dst_v)


@functools.partial(
    pl.kernel,
    out_type=(
        jax.ShapeDtypeStruct((NC, N, CH), jnp.float32),
        jax.ShapeDtypeStruct((NC, DEN_PAD), jnp.float32),
    ),
    mesh=_mesh,
    scratch_types=[
        pltpu.VMEM((N,), jnp.float32),
        pltpu.VMEM((N,), jnp.float32),
        pltpu.VMEM((L,), jnp.float32),
        pltpu.VMEM((CHUNK,), jnp.int32),
        pltpu.VMEM((CHUNK,), jnp.int32),
        pltpu.VMEM((CHUNK,), jnp.float32),
        pltpu.VMEM((CHUNK, CH), jnp.float32),
        pltpu.VMEM((ZROWS, CH), jnp.float32),
        pltpu.VMEM((DEN_PT,), jnp.float32),
        pltpu.VMEM_SHARED((N, CH), jnp.float32),
        pltpu.VMEM_SHARED((DEN_PAD,), jnp.float32),
        pltpu.SemaphoreType.DMA,
    ],
)
def _sc_edges(xp_hbm, asrc_hbm, adst_hbm, src_hbm, dst_hbm, cvec_hbm,
              outp_hbm, denp_hbm,
              asrc_v, adst_v, cvec_v, src_v, dst_v, ex_v, rows_v,
              zrow_v, zden_v, out_sh, den_sh, sem):
    cid = lax.axis_index("c")
    sid = lax.axis_index("s")
    wid = cid * NS + sid

    zf = jnp.zeros((L,), jnp.float32)

    @pl.loop(0, ZROWS)
    def _(r):
        for j in range(CH // L):
            zrow_v[r, pl.ds(j * L, L)] = zf

    @pl.loop(0, DEN_PT, step=L)
    def _(j):
        zden_v[pl.ds(j, L)] = zf

    @pl.loop(0, 5)
    def _(k):
        pltpu.sync_copy(zrow_v,
                        out_sh.at[pl.ds(sid * ROWS_PT + k * ZROWS, ZROWS)])

    pltpu.sync_copy(zden_v, den_sh.at[pl.ds(sid * DEN_PT, DEN_PT)])

    pltpu.sync_copy(asrc_hbm, asrc_v)
    pltpu.sync_copy(adst_hbm, adst_v)
    pltpu.sync_copy(cvec_hbm, cvec_v)

    plsc.subcore_barrier()

    cvec = cvec_v[...]
    base = wid * EPW

    @pl.loop(0, NCHUNK)
    def _(t):
        off = base + t * CHUNK
        pltpu.sync_copy(src_hbm.at[pl.ds(off, CHUNK)], src_v)
        pltpu.sync_copy(dst_hbm.at[pl.ds(off, CHUNK)], dst_v)
        gather = pltpu.async_copy(xp_hbm.at[src_v], rows_v, sem)

        @pl.loop(0, CHUNK, step=L)
        def _(j):
            sv = src_v[pl.ds(j, L)]
            dv = dst_v[pl.ds(j, L)]
            a = plsc.load_gather(asrc_v, [sv]) + plsc.load_gather(adst_v, [dv])
            a = jnp.where(a >= 0.0, a, 0.2 * a)
            ex_v[pl.ds(j, L)] = jnp.exp(a - cvec)

        pltpu.sync_copy(ex_v, den_sh.at[dst_v], add=True)
        gather.wait()

        @pl.loop(0, CHUNK)
        def _(e):
            s = plsc.load_gather(ex_v, [jnp.zeros((L,), jnp.int32) + e])
            for j in range(CH // L):
                rows_v[e, pl.ds(j * L, L)] = rows_v[e, pl.ds(j * L, L)] * s

        pltpu.sync_copy(rows_v, out_sh.at[dst_v], add=True)

    plsc.subcore_barrier()

    r0 = sid * ROWS_PT
    pltpu.sync_copy(out_sh.at[pl.ds(r0, ROWS_PT)],
                    outp_hbm.at[cid, pl.ds(r0, ROWS_PT)])
    d0 = sid * DEN_PT
    pltpu.sync_copy(den_sh.at[pl.ds(d0, DEN_PT)],
                    denp_hbm.at[cid, pl.ds(d0, DEN_PT)])


def _fin_body(o_ref, d_ref, b_ref, p_ref, out_ref):
    o = o_ref[0] + o_ref[1]
    den = d_ref[0] + d_ref[1]
    r = o / (den + 1e-16) + b_ref[...]
    out_ref[...] = jnp.where(r >= 0.0, r, p_ref[...] * r)


_FB = 500


def _finalize(outp, denp, bias2, prelu2):
    return pl.pallas_call(
        _fin_body,
        grid=(N // _FB,),
        in_specs=[
            pl.BlockSpec((NC, _FB, CH), lambda i: (0, i, 0)),
            pl.BlockSpec((NC, _FB, 1), lambda i: (0, i, 0)),
            pl.BlockSpec((1, CH), lambda i: (0, 0)),
            pl.BlockSpec((1, CH), lambda i: (0, 0)),
        ],
        out_specs=pl.BlockSpec((_FB, CH), lambda i: (i, 0)),
        out_shape=jax.ShapeDtypeStruct((N, CH), jnp.float32),
    )(outp, denp, bias2, prelu2)


def kernel(x, edge_index, W, a_src, a_dst, bias, prelu_w):
    src = edge_index[0].astype(jnp.int32)
    dst = edge_index[1].astype(jnp.int32)
    a_src_v = a_src.reshape(1, CH).astype(jnp.float32)
    a_dst_v = a_dst.reshape(1, CH).astype(jnp.float32)

    xp, asrc, adst, cmax = _project(x, W, a_src_v, a_dst_v)
    cvec = jnp.broadcast_to(cmax[0, 0], (L,))

    outp, denp = _sc_edges(xp, asrc[:, 0], adst[:, 0], src, dst, cvec)

    den = denp[:, :N].reshape(NC, N, 1)
    out = _finalize(outp, den, bias.reshape(1, CH), prelu_w.reshape(1, CH))
    return out


# speedup vs baseline: 22.4497x; 22.4497x over previous
import functools

import jax
import jax.numpy as jnp
from jax import lax
from jax.experimental import pallas as pl
from jax.experimental.pallas import tpu as pltpu
from jax.experimental.pallas import tpu_sc as plsc

N = 10000
E = 320000
CH = 128
NC, NS, L = 2, 16, 16
NW = NC * NS
EPW = E // NW
CHUNK = 80
NCHUNK = EPW // CHUNK
NPAD = 10240
ROWS_PT = NPAD // NS
DEN_PAD = 10240
DEN_PT = DEN_PAD // NS
ZROWS = 128

_mesh = plsc.VectorSubcoreMesh(
    core_axis_name="c", subcore_axis_name="s", num_cores=NC, num_subcores=NS
)


def _proj_body(x_ref, w_ref, asv_ref, adv_ref,
               xp_ref, as_ref, ad_ref, c_ref, ms_ref, md_ref):
    i = pl.program_id(0)
    xp = jnp.dot(x_ref[...], w_ref[...], preferred_element_type=jnp.float32)
    xp_ref[...] = xp
    s = jnp.sum(xp * asv_ref[...], axis=1, keepdims=True)
    d = jnp.sum(xp * adv_ref[...], axis=1, keepdims=True)
    as_ref[...] = s
    ad_ref[...] = d

    @pl.when(i == 0)
    def _():
        ms_ref[0] = -jnp.inf
        md_ref[0] = -jnp.inf

    ms_ref[0] = jnp.maximum(ms_ref[0], jnp.max(s))
    md_ref[0] = jnp.maximum(md_ref[0], jnp.max(d))

    @pl.when(i == pl.num_programs(0) - 1)
    def _():
        m = ms_ref[0] + md_ref[0]
        c_ref[...] = jnp.broadcast_to(jnp.where(m >= 0.0, m, 0.2 * m), (1, 1))


_PB = 400


def _project(x, W, a_src_v, a_dst_v):
    return pl.pallas_call(
        _proj_body,
        grid=(N // _PB,),
        in_specs=[
            pl.BlockSpec((_PB, CH), lambda i: (i, 0)),
            pl.BlockSpec((CH, CH), lambda i: (0, 0)),
            pl.BlockSpec((1, CH), lambda i: (0, 0)),
            pl.BlockSpec((1, CH), lambda i: (0, 0)),
        ],
        out_specs=[
            pl.BlockSpec((_PB, CH), lambda i: (i, 0)),
            pl.BlockSpec((_PB, 1), lambda i: (i, 0)),
            pl.BlockSpec((_PB, 1), lambda i: (i, 0)),
            pl.BlockSpec((1, 1), lambda i: (0, 0)),
        ],
        out_shape=[
            jax.ShapeDtypeStruct((N, CH), jnp.float32),
            jax.ShapeDtypeStruct((N, 1), jnp.float32),
            jax.ShapeDtypeStruct((N, 1), jnp.float32),
            jax.ShapeDtypeStruct((1, 1), jnp.float32),
        ],
        scratch_shapes=[
            pltpu.SMEM((1,), jnp.float32),
            pltpu.SMEM((1,), jnp.float32),
        ],
    )(x, W, a_src_v, a_dst_v)


@functools.partial(
    pl.kernel,
    out_type=(
        jax.ShapeDtypeStruct((NC, NPAD, CH), jnp.float32),
        jax.ShapeDtypeStruct((NC, DEN_PAD), jnp.float32),
    ),
    mesh=_mesh,
    scratch_types=[
        pltpu.VMEM((N,), jnp.float32),
        pltpu.VMEM((N,), jnp.float32),
        pltpu.VMEM((L,), jnp.float32),
        pltpu.VMEM((CHUNK,), jnp.int32),
        pltpu.VMEM((CHUNK,), jnp.int32),
        pltpu.VMEM((CHUNK,), jnp.float32),
        pltpu.VMEM((CHUNK, CH), jnp.float32),
        pltpu.VMEM((ZROWS, CH), jnp.float32),
        pltpu.VMEM((DEN_PT,), jnp.float32),
        pltpu.VMEM_SHARED((NPAD, CH), jnp.float32),
        pltpu.VMEM_SHARED((DEN_PAD,), jnp.float32),
        pltpu.SemaphoreType.DMA,
    ],
    compiler_params=pltpu.CompilerParams(needs_layout_passes=False),
)
def _sc_edges(xp_hbm, asrc_hbm, adst_hbm, src_hbm, dst_hbm, cvec_hbm,
              outp_hbm, denp_hbm,
              asrc_v, adst_v, cvec_v, src_v, dst_v, ex_v, rows_v,
              zrow_v, zden_v, out_sh, den_sh, sem):
    cid = lax.axis_index("c")
    sid = lax.axis_index("s")
    wid = cid * NS + sid

    zf = jnp.zeros((L,), jnp.float32)

    @pl.loop(0, ZROWS)
    def _(r):
        for j in range(CH // L):
            zrow_v[r, pl.ds(j * L, L)] = zf

    @pl.loop(0, DEN_PT, step=L)
    def _(j):
        zden_v[pl.ds(j, L)] = zf

    @pl.loop(0, 5)
    def _(k):
        pltpu.sync_copy(zrow_v,
                        out_sh.at[pl.ds(sid * ROWS_PT + k * ZROWS, ZROWS)])

    pltpu.sync_copy(zden_v, den_sh.at[pl.ds(sid * DEN_PT, DEN_PT)])

    pltpu.sync_copy(asrc_hbm, asrc_v)
    pltpu.sync_copy(adst_hbm, adst_v)
    pltpu.sync_copy(cvec_hbm, cvec_v)

    plsc.subcore_barrier()

    cvec = cvec_v[...]
    base = wid * EPW

    @pl.loop(0, NCHUNK)
    def _(t):
        off = base + t * CHUNK
        pltpu.sync_copy(src_hbm.at[pl.ds(off, CHUNK)], src_v)
        pltpu.sync_copy(dst_hbm.at[pl.ds(off, CHUNK)], dst_v)
        gather = pltpu.async_copy(xp_hbm.at[src_v], rows_v, sem)

        @pl.loop(0, CHUNK, step=L)
        def _(j):
            sv = src_v[pl.ds(j, L)]
            dv = dst_v[pl.ds(j, L)]
            a = plsc.load_gather(asrc_v, [sv]) + plsc.load_gather(adst_v, [dv])
            a = jnp.where(a >= 0.0, a, 0.2 * a)
            ex_v[pl.ds(j, L)] = jnp.exp(a - cvec)

        pltpu.sync_copy(ex_v, den_sh.at[dst_v], add=True)
        gather.wait()

        @pl.loop(0, CHUNK)
        def _(e):
            s = plsc.load_gather(ex_v, [jnp.zeros((L,), jnp.int32) + e])
            for j in range(CH // L):
                rows_v[e, pl.ds(j * L, L)] = rows_v[e, pl.ds(j * L, L)] * s

        pltpu.sync_copy(rows_v, out_sh.at[dst_v], add=True)

    plsc.subcore_barrier()

    r0 = sid * ROWS_PT
    pltpu.sync_copy(out_sh.at[pl.ds(r0, ROWS_PT)],
                    outp_hbm.at[cid, pl.ds(r0, ROWS_PT)])
    d0 = sid * DEN_PT
    pltpu.sync_copy(den_sh.at[pl.ds(d0, DEN_PT)],
                    denp_hbm.at[cid, pl.ds(d0, DEN_PT)])


def _fin_body(o_ref, d_ref, b_ref, p_ref, out_ref):
    o = o_ref[0] + o_ref[1]
    den = d_ref[0] + d_ref[1]
    r = o / (den + 1e-16) + b_ref[...]
    out_ref[...] = jnp.where(r >= 0.0, r, p_ref[...] * r)


_FB = 400


def _finalize(outp, denp, bias2, prelu2):
    return pl.pallas_call(
        _fin_body,
        grid=(N // _FB,),
        in_specs=[
            pl.BlockSpec((NC, _FB, CH), lambda i: (0, i, 0)),
            pl.BlockSpec((NC, _FB, 1), lambda i: (0, i, 0)),

            pl.BlockSpec((1, CH), lambda i: (0, 0)),
            pl.BlockSpec((1, CH), lambda i: (0, 0)),
        ],
        out_specs=pl.BlockSpec((_FB, CH), lambda i: (i, 0)),
        out_shape=jax.ShapeDtypeStruct((N, CH), jnp.float32),
    )(outp, denp, bias2, prelu2)


def kernel(x, edge_index, W, a_src, a_dst, bias, prelu_w):
    src = edge_index[0].astype(jnp.int32)
    dst = edge_index[1].astype(jnp.int32)
    a_src_v = a_src.reshape(1, CH).astype(jnp.float32)
    a_dst_v = a_dst.reshape(1, CH).astype(jnp.float32)

    xp, asrc, adst, cmax = _project(x, W, a_src_v, a_dst_v)
    cvec = jnp.broadcast_to(cmax[0, 0], (L,))

    outp, denp = _sc_edges(xp, asrc[:, 0], adst[:, 0], src, dst, cvec)

    den = denp.reshape(NC, DEN_PAD, 1)
    out = _finalize(outp, den, bias.reshape(1, CH), prelu_w.reshape(1, CH))
    return out
